# baseline (speedup 1.0000x reference)
import functools

import jax
import jax.numpy as jnp
from jax import lax
from jax.experimental import pallas as pl
from jax.experimental.pallas import tpu as pltpu


def _make_tc_gather(B, V, D):
    NQ = 16
    UNROLL = 16

    def body(ids_ref, table_ref, out_ref, *sems):
        def loop(t, carry):
            j0 = t * UNROLL
            for u in range(UNROLL):
                j = j0 + u
                i = ids_ref[j]
                pltpu.async_copy(
                    table_ref.at[pl.ds(i, 1)],
                    out_ref.at[pl.ds(j, 1)],
                    sems[u % NQ],
                )
            return carry

        lax.fori_loop(0, B // UNROLL, loop, 0)
        per_q = B // NQ
        for q in range(NQ):
            pltpu.make_async_copy(
                table_ref.at[pl.ds(0, per_q)],
                out_ref.at[pl.ds(q * per_q, per_q)],
                sems[q],
            ).wait()

    return pl.pallas_call(
        body,
        grid=(),
        in_specs=[
            pl.BlockSpec(memory_space=pltpu.SMEM),
            pl.BlockSpec(memory_space=pl.ANY),
        ],
        out_specs=pl.BlockSpec(memory_space=pl.ANY),
        out_shape=jax.ShapeDtypeStruct((B, D), jnp.float32),
        scratch_shapes=[pltpu.SemaphoreType.DMA] * NQ,
    )


def kernel(style_ids, table):
    (B,) = style_ids.shape
    V, D = table.shape
    return _make_tc_gather(B, V, D)(style_ids.astype(jnp.int32), table)


# speedup vs baseline: 1.2897x; 1.2897x over previous
import functools

import jax
import jax.numpy as jnp
from jax import lax
from jax.experimental import pallas as pl
from jax.experimental.pallas import tpu as pltpu
from jax.experimental.pallas import tpu_sc as plsc

_SC_SHARE = 10240


def _make_sc_gather(B, V, D):
    info = plsc.get_sparse_core_info()
    NC, NS = info.num_cores, info.num_subcores
    NW = NC * NS
    assert B % NW == 0
    b_per_w = B // NW
    NSEM = 8
    UNROLL = 8

    mesh = plsc.VectorSubcoreMesh(core_axis_name="c", subcore_axis_name="s")

    @functools.partial(
        pl.kernel,
        mesh=mesh,
        out_type=jax.ShapeDtypeStruct((B, D), jnp.float32),
        scratch_types=[
            pltpu.VMEM_SHARED((B,), jnp.int32),
            pltpu.SMEM((b_per_w,), jnp.int32),
            pltpu.VMEM((b_per_w, D), jnp.float32),
            [pltpu.SemaphoreType.DMA] * NSEM,
        ],
    )
    def k(ids_hbm, table_hbm, out_hbm, idx_sh, idx_s, rows_v, sems):
        wid = lax.axis_index("s") * NC + lax.axis_index("c")
        base = wid * b_per_w
        @pl.when(lax.axis_index("s") == 0)
        def _():
            pltpu.sync_copy(ids_hbm, idx_sh)
        plsc.subcore_barrier()
        pltpu.sync_copy(idx_sh.at[pl.ds(base, b_per_w)], idx_s)

        def body(t, carry):
            j0 = t * UNROLL
            for u in range(UNROLL):
                j = j0 + u
                i = idx_s[j]
                pltpu.async_copy(
                    table_hbm.at[pl.ds(i, 1)],
                    rows_v.at[pl.ds(j, 1)],
                    sems[u % NSEM],
                )
            return carry

        lax.fori_loop(0, b_per_w // UNROLL, body, 0)
        per_sem = b_per_w // NSEM
        for u in range(NSEM):
            pltpu.make_async_copy(
                table_hbm.at[pl.ds(0, per_sem)],
                rows_v.at[pl.ds(u * per_sem, per_sem)],
                sems[u],
            ).wait()
        pltpu.sync_copy(rows_v, out_hbm.at[pl.ds(base, b_per_w)])

    return k


def _make_tc_gather(B, V, D):
    NQ = 16
    UNROLL = 16

    def body(ids_ref, table_ref, out_ref, *sems):
        def loop(t, carry):
            j0 = t * UNROLL
            for u in range(UNROLL):
                j = j0 + u
                i = ids_ref[j]
                pltpu.async_copy(
                    table_ref.at[pl.ds(i, 1)],
                    out_ref.at[pl.ds(j, 1)],
                    sems[u % NQ],
                )
            return carry

        lax.fori_loop(0, B // UNROLL, loop, 0)
        per_q = B // NQ
        for q in range(NQ):
            pltpu.make_async_copy(
                table_ref.at[pl.ds(0, per_q)],
                out_ref.at[pl.ds(q * per_q, per_q)],
                sems[q],
            ).wait()

    return pl.pallas_call(
        body,
        grid=(),
        in_specs=[
            pl.BlockSpec(memory_space=pltpu.SMEM),
            pl.BlockSpec(memory_space=pl.ANY),
        ],
        out_specs=pl.BlockSpec(memory_space=pl.ANY),
        out_shape=jax.ShapeDtypeStruct((B, D), jnp.float32),
        scratch_shapes=[pltpu.SemaphoreType.DMA] * NQ,
    )


def kernel(style_ids, table):
    (B,) = style_ids.shape
    V, D = table.shape
    ids = style_ids.astype(jnp.int32)
    n_sc = _SC_SHARE
    sc_out = _make_sc_gather(n_sc, V, D)(ids[:n_sc], table)
    tc_out = _make_tc_gather(B - n_sc, V, D)(ids[n_sc:], table)
    return jnp.concatenate([sc_out, tc_out], axis=0)


# speedup vs baseline: 1.6347x; 1.2676x over previous
import functools

import jax
import jax.numpy as jnp
from jax import lax
from jax.experimental import pallas as pl
from jax.experimental.pallas import tpu as pltpu
from jax.experimental.pallas import tpu_sc as plsc

_SC_SHARE = 16384


def _make_sc_gather(B, V, D):
    info = plsc.get_sparse_core_info()
    NC, NS = info.num_cores, info.num_subcores
    NW = NC * NS
    assert B % NW == 0
    b_per_w = B // NW
    NSEM = 8
    UNROLL = 8

    mesh = plsc.VectorSubcoreMesh(core_axis_name="c", subcore_axis_name="s")

    @functools.partial(
        pl.kernel,
        mesh=mesh,
        out_type=jax.ShapeDtypeStruct((B, D), jnp.float32),
        scratch_types=[
            pltpu.VMEM_SHARED((B,), jnp.int32),
            pltpu.SMEM((b_per_w,), jnp.int32),
            pltpu.VMEM((b_per_w, D), jnp.float32),
            [pltpu.SemaphoreType.DMA] * NSEM,
        ],
    )
    def k(ids_hbm, table_hbm, out_hbm, idx_sh, idx_s, rows_v, sems):
        wid = lax.axis_index("s") * NC + lax.axis_index("c")
        base = wid * b_per_w
        @pl.when(lax.axis_index("s") == 0)
        def _():
            pltpu.sync_copy(ids_hbm, idx_sh)
        plsc.subcore_barrier()
        pltpu.sync_copy(idx_sh.at[pl.ds(base, b_per_w)], idx_s)

        def body(t, carry):
            j0 = t * UNROLL
            for u in range(UNROLL):
                j = j0 + u
                i = idx_s[j]
                pltpu.async_copy(
                    table_hbm.at[pl.ds(i, 1)],
                    rows_v.at[pl.ds(j, 1)],
                    sems[u % NSEM],
                )
            return carry

        lax.fori_loop(0, b_per_w // UNROLL, body, 0)
        per_sem = b_per_w // NSEM
        for u in range(NSEM):
            pltpu.make_async_copy(
                table_hbm.at[pl.ds(0, per_sem)],
                rows_v.at[pl.ds(u * per_sem, per_sem)],
                sems[u],
            ).wait()
        pltpu.sync_copy(rows_v, out_hbm.at[pl.ds(base, b_per_w)])

    return k


def _make_tc_gather(B, V, D):
    NQ = 16
    UNROLL = 16

    def body(ids_ref, table_ref, out_ref, *sems):
        def loop(t, carry):
            j0 = t * UNROLL
            for u in range(UNROLL):
                j = j0 + u
                i = ids_ref[j]
                pltpu.async_copy(
                    table_ref.at[pl.ds(i, 1)],
                    out_ref.at[pl.ds(j, 1)],
                    sems[u % NQ],
                )
            return carry

        lax.fori_loop(0, B // UNROLL, loop, 0)
        per_q = B // NQ
        for q in range(NQ):
            pltpu.make_async_copy(
                table_ref.at[pl.ds(0, per_q)],
                out_ref.at[pl.ds(q * per_q, per_q)],
                sems[q],
            ).wait()

    return pl.pallas_call(
        body,
        grid=(),
        in_specs=[
            pl.BlockSpec(memory_space=pltpu.SMEM),
            pl.BlockSpec(memory_space=pl.ANY),
        ],
        out_specs=pl.BlockSpec(memory_space=pl.ANY),
        out_shape=jax.ShapeDtypeStruct((B, D), jnp.float32),
        scratch_shapes=[pltpu.SemaphoreType.DMA] * NQ,
    )


def kernel(style_ids, table):
    (B,) = style_ids.shape
    V, D = table.shape
    ids = style_ids.astype(jnp.int32)
    n_sc = _SC_SHARE
    sc_out = _make_sc_gather(n_sc, V, D)(ids[:n_sc], table)
    if n_sc == B:
        return sc_out
    tc_out = _make_tc_gather(B - n_sc, V, D)(ids[n_sc:], table)
    return jnp.concatenate([sc_out, tc_out], axis=0)
